# baseline (speedup 1.0000x reference)
import functools

import jax
import jax.numpy as jnp
from jax import lax
from jax.experimental import pallas as pl
from jax.experimental.pallas import tpu as pltpu
from jax.experimental.pallas import tpu_sc as plsc


@functools.lru_cache(maxsize=None)
def _make_broadcast_copy(batch: int, seq_len: int, hidden: int):
    info = plsc.get_sparse_core_info()
    num_workers = info.num_cores * info.num_subcores
    assert seq_len % num_workers == 0
    rows_per_w = seq_len // num_workers

    mesh = plsc.ScalarSubcoreMesh(axis_name="c")
    rows_per_w = seq_len // info.num_cores

    n_chunks = 8
    assert rows_per_w % n_chunks == 0
    rows_per_c = rows_per_w // n_chunks

    @functools.partial(
        pl.kernel,
        mesh=mesh,
        out_type=jax.ShapeDtypeStruct((batch, seq_len, hidden), jnp.float32),
        scratch_types=[
            pltpu.VMEM_SHARED((n_chunks, rows_per_c, hidden), jnp.float32),
            pltpu.SemaphoreType.DMA,
            pltpu.SemaphoreType.DMA,
        ],
    )
    def broadcast_copy(table_hbm, out_hbm, buf, rsem, wsem):
        wid = lax.axis_index("c")
        base = wid * rows_per_w
        reads = [
            pltpu.async_copy(
                table_hbm.at[pl.ds(base + c * rows_per_c, rows_per_c)],
                buf.at[c], rsem)
            for c in range(n_chunks)
        ]
        writes = []
        for c in range(n_chunks):
            reads[c].wait()
            writes += [
                pltpu.async_copy(
                    buf.at[c],
                    out_hbm.at[b, pl.ds(base + c * rows_per_c, rows_per_c)],
                    wsem)
                for b in range(batch)
            ]
        for w in writes:
            w.wait()

    return broadcast_copy


def kernel(hidden_size, table):
    batch, seq_len, _ = hidden_size.shape
    hidden = table.shape[1]
    return _make_broadcast_copy(batch, seq_len, hidden)(table)


# speedup vs baseline: 1.1525x; 1.1525x over previous
import functools

import jax
import jax.numpy as jnp
from jax import lax
from jax.experimental import pallas as pl
from jax.experimental.pallas import tpu as pltpu
from jax.experimental.pallas import tpu_sc as plsc


@functools.lru_cache(maxsize=None)
def _make_broadcast_copy(batch: int, seq_len: int, hidden: int):
    info = plsc.get_sparse_core_info()
    num_workers = info.num_cores * info.num_subcores
    assert seq_len % num_workers == 0
    rows_per_w = seq_len // num_workers

    mesh = plsc.VectorSubcoreMesh(core_axis_name="c", subcore_axis_name="s")

    n_chunks = 4
    assert rows_per_w % n_chunks == 0
    rows_per_c = rows_per_w // n_chunks

    @functools.partial(
        pl.kernel,
        mesh=mesh,
        out_type=jax.ShapeDtypeStruct((batch, seq_len, hidden), jnp.float32),
        scratch_types=[
            pltpu.VMEM((n_chunks, rows_per_c, hidden), jnp.float32),
            pltpu.SemaphoreType.DMA,
            pltpu.SemaphoreType.DMA,
        ],
    )
    def broadcast_copy(table_hbm, out_hbm, buf, rsem, wsem):
        wid = lax.axis_index("c") * info.num_subcores + lax.axis_index("s")
        base = wid * rows_per_w
        reads = [
            pltpu.async_copy(
                table_hbm.at[pl.ds(base + c * rows_per_c, rows_per_c)],
                buf.at[c], rsem)
            for c in range(n_chunks)
        ]
        writes = []
        for c in range(n_chunks):
            reads[c].wait()
            writes += [
                pltpu.async_copy(
                    buf.at[c],
                    out_hbm.at[b, pl.ds(base + c * rows_per_c, rows_per_c)],
                    wsem)
                for b in range(batch)
            ]
        for w in writes:
            w.wait()

    return broadcast_copy


def kernel(hidden_size, table):
    batch, seq_len, _ = hidden_size.shape
    hidden = table.shape[1]
    return _make_broadcast_copy(batch, seq_len, hidden)(table)
